# baseline (speedup 1.0000x reference)
import numpy as np
import jax
import jax.numpy as jnp
from jax.experimental import pallas as pl
from jax.experimental.pallas import tpu as pltpu


def _line_constants(num_rows: int):
    rng = np.random.default_rng(0)
    rand_row = int(rng.integers(0, num_rows))
    x_min, alpha = 1.0728769e-07, 1.0868737
    r = float(rng.random())
    relative_error = x_min * (1.0 - r) ** (-1.0 / (alpha - 1.0))
    return rand_row, relative_error


_CHUNK_ROWS = 2048


def kernel(forward_input):
    n_rows, n_cols = forward_input.shape
    rand_row, rel_err = _line_constants(n_rows)

    grp = rand_row - (rand_row % 8)
    off = rand_row - grp

    segs = []

    def _add_range(a, b):
        s = a
        while s < b:
            e = min(s + _CHUNK_ROWS, b)
            segs.append((s, e - s))
            s = e

    _add_range(0, grp)
    _add_range(grp + 8, n_rows)
    n_segs = len(segs)

    def body(x_hbm, o_hbm, row_v, sems, row_sem):
        copies = []
        for i, (s, sz) in enumerate(segs):
            cp = pltpu.make_async_copy(
                x_hbm.at[pl.ds(s, sz)], o_hbm.at[pl.ds(s, sz)], sems.at[i]
            )
            cp.start()
            copies.append(cp)

        rin = pltpu.make_async_copy(x_hbm.at[pl.ds(grp, 8)], row_v, row_sem)
        rin.start()
        rin.wait()
        row_v[off, :] = row_v[off, :] * jnp.float32(rel_err)
        rout = pltpu.make_async_copy(row_v, o_hbm.at[pl.ds(grp, 8)], row_sem)
        rout.start()

        for cp in copies:
            cp.wait()
        rout.wait()

    return pl.pallas_call(
        body,
        in_specs=[pl.BlockSpec(memory_space=pl.ANY)],
        out_specs=pl.BlockSpec(memory_space=pl.ANY),
        out_shape=jax.ShapeDtypeStruct((n_rows, n_cols), forward_input.dtype),
        scratch_shapes=[
            pltpu.VMEM((8, n_cols), forward_input.dtype),
            pltpu.SemaphoreType.DMA((n_segs,)),
            pltpu.SemaphoreType.DMA,
        ],
    )(forward_input)


# speedup vs baseline: 47.1799x; 47.1799x over previous
import numpy as np
import jax
import jax.numpy as jnp
from jax.experimental import pallas as pl
from jax.experimental.pallas import tpu as pltpu


def _line_constants(num_rows: int):
    rng = np.random.default_rng(0)
    rand_row = int(rng.integers(0, num_rows))
    x_min, alpha = 1.0728769e-07, 1.0868737
    r = float(rng.random())
    relative_error = x_min * (1.0 - r) ** (-1.0 / (alpha - 1.0))
    return rand_row, relative_error


_BLOCK_ROWS = 1024


def kernel(forward_input):
    n_rows, n_cols = forward_input.shape
    rand_row, rel_err = _line_constants(n_rows)

    block_rows = _BLOCK_ROWS
    grid = n_rows // block_rows
    target_block = rand_row // block_rows
    row_off = rand_row % block_rows

    def body(x_ref, o_ref):
        i = pl.program_id(0)
        o_ref[...] = x_ref[...]

        @pl.when(i == target_block)
        def _():
            o_ref[row_off, :] = x_ref[row_off, :] * jnp.float32(rel_err)

    return pl.pallas_call(
        body,
        grid=(grid,),
        in_specs=[pl.BlockSpec((block_rows, n_cols), lambda i: (i, 0))],
        out_specs=pl.BlockSpec((block_rows, n_cols), lambda i: (i, 0)),
        out_shape=jax.ShapeDtypeStruct((n_rows, n_cols), forward_input.dtype),
    )(forward_input)


# speedup vs baseline: 48.8883x; 1.0362x over previous
import numpy as np
import jax
import jax.numpy as jnp
from jax.experimental import pallas as pl
from jax.experimental.pallas import tpu as pltpu


def _line_constants(num_rows: int):
    rng = np.random.default_rng(0)
    rand_row = int(rng.integers(0, num_rows))
    x_min, alpha = 1.0728769e-07, 1.0868737
    r = float(rng.random())
    relative_error = x_min * (1.0 - r) ** (-1.0 / (alpha - 1.0))
    return rand_row, relative_error


_BLOCK_ROWS = 2048


def kernel(forward_input):
    n_rows, n_cols = forward_input.shape
    rand_row, rel_err = _line_constants(n_rows)

    block_rows = _BLOCK_ROWS
    grid = n_rows // block_rows
    target_block = rand_row // block_rows
    row_off = rand_row % block_rows

    def body(x_ref, o_ref):
        i = pl.program_id(0)
        o_ref[...] = x_ref[...]

        @pl.when(i == target_block)
        def _():
            o_ref[row_off, :] = x_ref[row_off, :] * jnp.float32(rel_err)

    return pl.pallas_call(
        body,
        grid=(grid,),
        in_specs=[pl.BlockSpec((block_rows, n_cols), lambda i: (i, 0))],
        out_specs=pl.BlockSpec((block_rows, n_cols), lambda i: (i, 0)),
        out_shape=jax.ShapeDtypeStruct((n_rows, n_cols), forward_input.dtype),
    )(forward_input)
